# baseline (speedup 1.0000x reference)
import functools

import jax
import jax.numpy as jnp
from jax import lax
from jax.experimental import pallas as pl
from jax.experimental.pallas import tpu as pltpu
from jax.experimental.pallas import tpu_sc as plsc

N = 10000
E = 160000
H = 512
R = 3

RN = 105
NRANGES = 96
RPT = 3
NPAD = NRANGES * RN
KB = 32
CHUNK = 2000
NCHUNK = E // CHUNK
CVREGS = CHUNK // 16

_sc_mesh = plsc.VectorSubcoreMesh(core_axis_name="c", subcore_axis_name="s")


@functools.partial(
    pl.kernel,
    mesh=_sc_mesh,
    out_type=[
        jax.ShapeDtypeStruct((NPAD * H,), jnp.float32),
        jax.ShapeDtypeStruct((NPAD * H,), jnp.float32),
    ],
    scratch_types=[
        pltpu.VMEM((RN * H,), jnp.float32),
        pltpu.VMEM((RN * H,), jnp.float32),
        pltpu.VMEM((KB, H), jnp.float32),
        pltpu.VMEM((KB + 16,), jnp.int32),
        pltpu.VMEM((KB + 16,), jnp.int32),
        pltpu.VMEM((KB,), jnp.int32),
        pltpu.VMEM((KB + 16,), jnp.int32),
        pltpu.VMEM((CHUNK,), jnp.int32),
        pltpu.VMEM((CHUNK,), jnp.int32),
        pltpu.SemaphoreType.DMA,
    ],
    compiler_params=pltpu.CompilerParams(needs_layout_passes=False),
)
def _sc_segment_maxsum(x_hbm, src_hbm, dst_hbm, mx_hbm, sm_hbm,
                       accmx, accsm, rows, psrc, pldst, gidx, lsnap,
                       csrc, cdst, gsem):
    wid = lax.axis_index("s") * 2 + lax.axis_index("c")
    lanes = lax.iota(jnp.int32, 16)

    for j in range((KB + 16) // 16):
        psrc[pl.ds(j * 16, 16)] = lanes + j * 16
        pldst[pl.ds(j * 16, 16)] = lanes
        lsnap[pl.ds(j * 16, 16)] = lanes
    for j in range(KB // 16):
        gidx[pl.ds(j * 16, 16)] = lanes + j * 16

    def maxsum_edges(nedges, dyn):
        def edge_body(e, carry):
            off = lsnap[pl.ds(e, 16)][0] * H
            for c in range(H // 16):
                rv = rows[e, pl.ds(c * 16, 16)]
                f = c * 16
                mv = accmx[pl.ds(off + f, 16)]
                accmx[pl.ds(off + f, 16)] = jnp.maximum(mv, rv)
                plsc.addupdate_scatter(accsm, [lanes + (off + f)], rv)
            return carry
        if dyn:
            lax.fori_loop(0, nedges, edge_body, 0)
        else:
            lax.fori_loop(0, KB, edge_body, 0, unroll=2)

    def drain_pending():
        pltpu.make_async_copy(x_hbm.at[gidx], rows, gsem).wait()
        maxsum_edges(KB, dyn=False)

    def snapshot(self_=None):
        for j in range(KB // 16):
            gidx[pl.ds(j * 16, 16)] = psrc[pl.ds(j * 16, 16)]
            lsnap[pl.ds(j * 16, 16)] = pldst[pl.ds(j * 16, 16)]

    def flush(carry):
        n, fc = carry
        snapshot()
        pltpu.async_copy(x_hbm.at[gidx], rows, gsem).wait()
        maxsum_edges(KB, dyn=False)
        psrc[pl.ds(0, 16)] = psrc[pl.ds(KB, 16)]
        pldst[pl.ds(0, 16)] = pldst[pl.ds(KB, 16)]
        return (n - KB, fc + 1)

    for k in range(RPT):
        lo = (wid * RPT + k) * RN

        def init_body(i, carry):
            accmx[pl.ds(i * 16, 16)] = jnp.full((16,), -jnp.inf, jnp.float32)
            accsm[pl.ds(i * 16, 16)] = jnp.zeros((16,), jnp.float32)
            return carry
        lax.fori_loop(0, RN * H // 16, init_body, 0)

        def superchunk(s, carry):
            pltpu.sync_copy(src_hbm.at[pl.ds(s * CHUNK, CHUNK)], csrc)
            pltpu.sync_copy(dst_hbm.at[pl.ds(s * CHUNK, CHUNK)], cdst)

            def chunk(j, carry):
                n, fc = carry
                s16 = csrc[pl.ds(j * 16, 16)]
                ld = cdst[pl.ds(j * 16, 16)] - lo
                m = (ld >= 0) & (ld < RN)
                plsc.store_compressed(psrc.at[pl.ds(n, 16)], s16, mask=m)
                plsc.store_compressed(pldst.at[pl.ds(n, 16)], ld, mask=m)
                n = n + jnp.sum(m.astype(jnp.int32))
                return lax.cond(n >= KB, flush, lambda c: c, (n, fc))

            return lax.fori_loop(0, CVREGS, chunk, carry)

        n, fc = lax.fori_loop(0, NCHUNK, superchunk, (0, 0))

        for j in range(KB // 16):
            gidx[pl.ds(j * 16, 16)] = psrc[pl.ds(j * 16, 16)]
            lsnap[pl.ds(j * 16, 16)] = pldst[pl.ds(j * 16, 16)]
        pltpu.async_copy(x_hbm.at[gidx], rows, gsem).wait()
        maxsum_edges(n, dyn=True)

        def fin_body(i, carry):
            v = accmx[pl.ds(i * 16, 16)]
            accmx[pl.ds(i * 16, 16)] = jnp.where(
                v == -jnp.inf, jnp.zeros((16,), jnp.float32), v)
            return carry
        lax.fori_loop(0, RN * H // 16, fin_body, 0)

        pltpu.sync_copy(accmx, mx_hbm.at[pl.ds(lo * H, RN * H)])
        pltpu.sync_copy(accsm, sm_hbm.at[pl.ds(lo * H, RN * H)])


BM = 400
GRID = N // BM


def _embed_body(xu_ref, w_ref, b_ref, o_ref):
    o_ref[...] = jnp.dot(xu_ref[...].astype(jnp.bfloat16), w_ref[...],
                         preferred_element_type=jnp.float32) + b_ref[...]


def _tc_embed(xu, w_bf16, b):
    return pl.pallas_call(
        _embed_body,
        grid=(GRID,),
        in_specs=[pl.BlockSpec((BM, H), lambda i: (i, 0)),
                  pl.BlockSpec((H, H), lambda i: (0, 0)),
                  pl.BlockSpec((1, H), lambda i: (0, 0))],
        out_specs=pl.BlockSpec((BM, H), lambda i: (i, 0)),
        out_shape=jax.ShapeDtypeStruct((N, H), jnp.float32),
    )(xu, w_bf16, b)


def _conv_body(x_ref, mx_ref, sm_ref, w2ab_ref, w2c_ref, b2_ref,
               w1_ref, b1_ref, w3a_ref, w3b_ref, b3_ref, o_ref):
    a = (jnp.dot(mx_ref[...].astype(jnp.bfloat16), w2ab_ref[...],
                 preferred_element_type=jnp.float32)
         + jnp.dot(sm_ref[...].astype(jnp.bfloat16), w2c_ref[...],
                   preferred_element_type=jnp.float32)
         + b2_ref[...])
    bb = jnp.dot(x_ref[...].astype(jnp.bfloat16), w1_ref[...],
                 preferred_element_type=jnp.float32) + b1_ref[...]
    a = jnp.maximum(a, 0.0).astype(jnp.bfloat16)
    bb = jnp.maximum(bb, 0.0).astype(jnp.bfloat16)
    h = (jnp.dot(a, w3a_ref[...], preferred_element_type=jnp.float32)
         + jnp.dot(bb, w3b_ref[...], preferred_element_type=jnp.float32)
         + b3_ref[...])
    o_ref[...] = jnp.maximum(h, 0.0)


def _tc_conv(x, mx, sm, wp):
    return pl.pallas_call(
        _conv_body,
        grid=(GRID,),
        in_specs=[pl.BlockSpec((BM, H), lambda i: (i, 0)),
                  pl.BlockSpec((BM, H), lambda i: (i, 0)),
                  pl.BlockSpec((BM, H), lambda i: (i, 0))]
                 + [pl.BlockSpec((H, H), lambda i: (0, 0)),
                    pl.BlockSpec((H, H), lambda i: (0, 0)),
                    pl.BlockSpec((1, H), lambda i: (0, 0)),
                    pl.BlockSpec((H, H), lambda i: (0, 0)),
                    pl.BlockSpec((1, H), lambda i: (0, 0)),
                    pl.BlockSpec((H, H), lambda i: (0, 0)),
                    pl.BlockSpec((H, H), lambda i: (0, 0)),
                    pl.BlockSpec((1, H), lambda i: (0, 0))],
        out_specs=pl.BlockSpec((BM, H), lambda i: (i, 0)),
        out_shape=jax.ShapeDtypeStruct((N, H), jnp.float32),
    )(x, mx, sm, *wp)


def _attn_body(h0_ref, h1_ref, h2_ref, p1_ref, b1_ref, o_ref):
    @pl.when(pl.program_id(0) == 0)
    def _():
        o_ref[...] = jnp.zeros_like(o_ref)
    for r, href in enumerate((h0_ref, h1_ref, h2_ref)):
        t = jnp.tanh(jnp.dot(href[...].astype(jnp.bfloat16), p1_ref[...],
                             preferred_element_type=jnp.float32) + b1_ref[...])
        o_ref[pl.ds(r, 1), :] = o_ref[pl.ds(r, 1), :] + jnp.sum(t, axis=0, keepdims=True)


def _tc_attn_colsums(h0, h1, h2, p1_bf16, b1):
    return pl.pallas_call(
        _attn_body,
        grid=(GRID,),
        in_specs=[pl.BlockSpec((BM, H), lambda i: (i, 0)),
                  pl.BlockSpec((BM, H), lambda i: (i, 0)),
                  pl.BlockSpec((BM, H), lambda i: (i, 0)),
                  pl.BlockSpec((H, H), lambda i: (0, 0)),
                  pl.BlockSpec((1, H), lambda i: (0, 0))],
        out_specs=pl.BlockSpec((R, H), lambda i: (0, 0)),
        out_shape=jax.ShapeDtypeStruct((R, H), jnp.float32),
    )(h0, h1, h2, p1_bf16, b1)


def _combine_body(h0_ref, h1_ref, h2_ref, beta_ref, o_ref):
    b = beta_ref[...]
    o_ref[...] = jnp.maximum(
        b[0, 0] * h0_ref[...] + b[0, 1] * h1_ref[...] + b[0, 2] * h2_ref[...],
        0.0)


def _tc_combine_relu(h0, h1, h2, beta):
    return pl.pallas_call(
        _combine_body,
        grid=(GRID,),
        in_specs=[pl.BlockSpec((BM, H), lambda i: (i, 0)),
                  pl.BlockSpec((BM, H), lambda i: (i, 0)),
                  pl.BlockSpec((BM, H), lambda i: (i, 0)),
                  pl.BlockSpec((1, R), lambda i: (0, 0))],
        out_specs=pl.BlockSpec((BM, H), lambda i: (i, 0)),
        out_shape=jax.ShapeDtypeStruct((N, H), jnp.float32),
    )(h0, h1, h2, beta)


def _pred_body(h0_ref, h1_ref, h2_ref, beta_ref, pw_ref, pb_ref, o_ref):
    b = beta_ref[...]
    h = b[0, 0] * h0_ref[...] + b[0, 1] * h1_ref[...] + b[0, 2] * h2_ref[...]
    logit = jnp.sum(h * pw_ref[...], axis=1, keepdims=True) + pb_ref[...]
    o_ref[...] = jax.nn.sigmoid(logit)


def _tc_combine_pred(h0, h1, h2, beta, pw_row, pb):
    return pl.pallas_call(
        _pred_body,
        grid=(GRID,),
        in_specs=[pl.BlockSpec((BM, H), lambda i: (i, 0)),
                  pl.BlockSpec((BM, H), lambda i: (i, 0)),
                  pl.BlockSpec((BM, H), lambda i: (i, 0)),
                  pl.BlockSpec((1, R), lambda i: (0, 0)),
                  pl.BlockSpec((1, H), lambda i: (0, 0)),
                  pl.BlockSpec((1, 1), lambda i: (0, 0))],
        out_specs=pl.BlockSpec((BM, 1), lambda i: (i, 0)),
        out_shape=jax.ShapeDtypeStruct((N, 1), jnp.float32),
    )(h0, h1, h2, beta, pw_row, pb)


def _conv_weights(p):
    w2 = p["fc2_W"]
    return (
        (w2[:H] + w2[H:2 * H]).astype(jnp.bfloat16),
        w2[2 * H:].astype(jnp.bfloat16),
        p["fc2_b"].reshape(1, H),
        p["fc1_W"].astype(jnp.bfloat16),
        p["fc1_b"].reshape(1, H),
        p["fc3_W"][:H].astype(jnp.bfloat16),
        p["fc3_W"][H:].astype(jnp.bfloat16),
        p["fc3_b"].reshape(1, H),
    )


def _layer(x2d, edges, convs, attn_p1, attn_b1, attn_p2):
    hs = []
    for r in range(R):
        mx, sm = _sc_segment_maxsum(x2d, edges[r][0], edges[r][1])
        mx = mx.reshape(NPAD, H)
        sm = sm.reshape(NPAD, H)
        hs.append(_tc_conv(x2d, mx, sm, convs[r]))
    colsums = _tc_attn_colsums(hs[0], hs[1], hs[2], attn_p1, attn_b1)
    w = (colsums @ attn_p2) / N
    beta = jax.nn.softmax(w, axis=0).reshape(1, R)
    return hs, beta


def kernel(x_user, params, edge_index_b0_r0, edge_index_b0_r1, edge_index_b0_r2,
           edge_index_b1_r0, edge_index_b1_r1, edge_index_b1_r2):
    edges0 = [edge_index_b0_r0, edge_index_b0_r1, edge_index_b0_r2]
    edges1 = [edge_index_b1_r0, edge_index_b1_r1, edge_index_b1_r2]

    x = _tc_embed(x_user, params["embed_W"].astype(jnp.bfloat16),
                  params["embed_b"].reshape(1, H))

    convs1 = [_conv_weights(params["conv1_r%d" % r]) for r in range(R)]
    convs2 = [_conv_weights(params["conv2_r%d" % r]) for r in range(R)]
    attn_p1 = params["attn_p1_W"].astype(jnp.bfloat16)
    attn_b1 = params["attn_p1_b"].reshape(1, H)
    attn_p2 = params["attn_p2_W"]

    hs1, beta1 = _layer(x, edges0, convs1, attn_p1, attn_b1, attn_p2)
    h = _tc_combine_relu(hs1[0], hs1[1], hs1[2], beta1)
    hs2, beta2 = _layer(h, edges1, convs2, attn_p1, attn_b1, attn_p2)
    return _tc_combine_pred(hs2[0], hs2[1], hs2[2], beta2,
                            params["pred_W"].reshape(1, H),
                            params["pred_b"].reshape(1, 1))


# speedup vs baseline: 1.0775x; 1.0775x over previous
import functools

import jax
import jax.numpy as jnp
from jax import lax
from jax.experimental import pallas as pl
from jax.experimental.pallas import tpu as pltpu
from jax.experimental.pallas import tpu_sc as plsc

N = 10000
E = 160000
H = 512
R = 3

RN = 105
NRANGES = 96
RPT = 3
NPAD = NRANGES * RN
KB = 32
CHUNK = 2000
NCHUNK = E // CHUNK
CVREGS = CHUNK // 16

_sc_mesh = plsc.VectorSubcoreMesh(core_axis_name="c", subcore_axis_name="s")


@functools.partial(
    pl.kernel,
    mesh=_sc_mesh,
    out_type=[
        jax.ShapeDtypeStruct((NPAD * H,), jnp.float32),
        jax.ShapeDtypeStruct((NPAD * H,), jnp.float32),
    ],
    scratch_types=[
        pltpu.VMEM((RN * H,), jnp.float32),
        pltpu.VMEM((RN * H,), jnp.float32),
        pltpu.VMEM((KB, H), jnp.float32),
        pltpu.VMEM((KB + 16,), jnp.int32),
        pltpu.VMEM((KB + 16,), jnp.int32),
        pltpu.VMEM((KB,), jnp.int32),
        pltpu.VMEM((KB + 16,), jnp.int32),
        pltpu.VMEM((CHUNK,), jnp.int32),
        pltpu.VMEM((CHUNK,), jnp.int32),
        pltpu.SemaphoreType.DMA,
    ],
    compiler_params=pltpu.CompilerParams(needs_layout_passes=False),
)
def _sc_segment_maxsum(x_hbm, src_hbm, dst_hbm, mx_hbm, sm_hbm,
                       accmx, accsm, rows, psrc, pldst, gidx, lsnap,
                       csrc, cdst, gsem):
    wid = lax.axis_index("s") * 2 + lax.axis_index("c")
    lanes = lax.iota(jnp.int32, 16)

    for j in range((KB + 16) // 16):
        psrc[pl.ds(j * 16, 16)] = lax.shift_left(lanes + j * 16, 7)
    for j in range(KB // 16):
        gidx[pl.ds(j * 16, 16)] = lanes + j * 16
        lsnap[pl.ds(j * 16, 16)] = lanes

    def maxsum_edges(nedges, dyn):
        def edge_body(e, carry):
            off = lsnap[pl.ds(e, 16)][0] * H
            for c in range(H // 16):
                rv = rows[e, pl.ds(c * 16, 16)]
                f = c * 16
                mv = accmx[pl.ds(off + f, 16)]
                accmx[pl.ds(off + f, 16)] = jnp.maximum(mv, rv)
                plsc.addupdate_scatter(accsm, [lanes + (off + f)], rv)
            return carry
        if dyn:
            lax.fori_loop(0, nedges, edge_body, 0)
        else:
            lax.fori_loop(0, KB, edge_body, 0, unroll=2)

    def drain_pending():
        pltpu.make_async_copy(x_hbm.at[gidx], rows, gsem).wait()
        maxsum_edges(KB, dyn=False)

    def snapshot(self_=None):
        for j in range(KB // 16):
            v = psrc[pl.ds(j * 16, 16)]
            gidx[pl.ds(j * 16, 16)] = lax.shift_right_logical(v, 7)
            lsnap[pl.ds(j * 16, 16)] = v & 127

    def flush(carry):
        n, fc = carry
        snapshot()
        pltpu.async_copy(x_hbm.at[gidx], rows, gsem).wait()
        maxsum_edges(KB, dyn=False)
        psrc[pl.ds(0, 16)] = psrc[pl.ds(KB, 16)]
        return (n - KB, fc + 1)

    for k in range(RPT):
        lo = (wid * RPT + k) * RN

        def init_body(i, carry):
            accmx[pl.ds(i * 16, 16)] = jnp.full((16,), -jnp.inf, jnp.float32)
            accsm[pl.ds(i * 16, 16)] = jnp.zeros((16,), jnp.float32)
            return carry
        lax.fori_loop(0, RN * H // 16, init_body, 0)

        def superchunk(s, carry):
            pltpu.sync_copy(src_hbm.at[pl.ds(s * CHUNK, CHUNK)], csrc)
            pltpu.sync_copy(dst_hbm.at[pl.ds(s * CHUNK, CHUNK)], cdst)

            def chunk(j, carry):
                n, fc = carry
                s16 = csrc[pl.ds(j * 16, 16)]
                ld = cdst[pl.ds(j * 16, 16)] - lo
                m = (ld >= 0) & (ld < RN)
                plsc.store_compressed(
                    psrc.at[pl.ds(n, 16)],
                    lax.shift_left(s16, 7) + ld, mask=m)
                n = n + plsc.all_reduce_population_count(m)[0]
                return lax.cond(n >= KB, flush, lambda c: c, (n, fc))

            return lax.fori_loop(0, CVREGS, chunk, carry, unroll=2)

        n, fc = lax.fori_loop(0, NCHUNK, superchunk, (0, 0))

        snapshot()
        pltpu.async_copy(x_hbm.at[gidx], rows, gsem).wait()
        maxsum_edges(n, dyn=True)

        def fin_body(i, carry):
            v = accmx[pl.ds(i * 16, 16)]
            accmx[pl.ds(i * 16, 16)] = jnp.where(
                v == -jnp.inf, jnp.zeros((16,), jnp.float32), v)
            return carry
        lax.fori_loop(0, RN * H // 16, fin_body, 0)

        pltpu.sync_copy(accmx, mx_hbm.at[pl.ds(lo * H, RN * H)])
        pltpu.sync_copy(accsm, sm_hbm.at[pl.ds(lo * H, RN * H)])


BM = 400
GRID = N // BM


def _embed_body(xu_ref, w_ref, b_ref, o_ref):
    o_ref[...] = jnp.dot(xu_ref[...].astype(jnp.bfloat16), w_ref[...],
                         preferred_element_type=jnp.float32) + b_ref[...]


def _tc_embed(xu, w_bf16, b):
    return pl.pallas_call(
        _embed_body,
        grid=(GRID,),
        in_specs=[pl.BlockSpec((BM, H), lambda i: (i, 0)),
                  pl.BlockSpec((H, H), lambda i: (0, 0)),
                  pl.BlockSpec((1, H), lambda i: (0, 0))],
        out_specs=pl.BlockSpec((BM, H), lambda i: (i, 0)),
        out_shape=jax.ShapeDtypeStruct((N, H), jnp.float32),
    )(xu, w_bf16, b)


def _conv_body(x_ref, mx_ref, sm_ref, w2ab_ref, w2c_ref, b2_ref,
               w1_ref, b1_ref, w3a_ref, w3b_ref, b3_ref, o_ref):
    a = (jnp.dot(mx_ref[...].astype(jnp.bfloat16), w2ab_ref[...],
                 preferred_element_type=jnp.float32)
         + jnp.dot(sm_ref[...].astype(jnp.bfloat16), w2c_ref[...],
                   preferred_element_type=jnp.float32)
         + b2_ref[...])
    bb = jnp.dot(x_ref[...].astype(jnp.bfloat16), w1_ref[...],
                 preferred_element_type=jnp.float32) + b1_ref[...]
    a = jnp.maximum(a, 0.0).astype(jnp.bfloat16)
    bb = jnp.maximum(bb, 0.0).astype(jnp.bfloat16)
    h = (jnp.dot(a, w3a_ref[...], preferred_element_type=jnp.float32)
         + jnp.dot(bb, w3b_ref[...], preferred_element_type=jnp.float32)
         + b3_ref[...])
    o_ref[...] = jnp.maximum(h, 0.0)


def _tc_conv(x, mx, sm, wp):
    return pl.pallas_call(
        _conv_body,
        grid=(GRID,),
        in_specs=[pl.BlockSpec((BM, H), lambda i: (i, 0)),
                  pl.BlockSpec((BM, H), lambda i: (i, 0)),
                  pl.BlockSpec((BM, H), lambda i: (i, 0))]
                 + [pl.BlockSpec((H, H), lambda i: (0, 0)),
                    pl.BlockSpec((H, H), lambda i: (0, 0)),
                    pl.BlockSpec((1, H), lambda i: (0, 0)),
                    pl.BlockSpec((H, H), lambda i: (0, 0)),
                    pl.BlockSpec((1, H), lambda i: (0, 0)),
                    pl.BlockSpec((H, H), lambda i: (0, 0)),
                    pl.BlockSpec((H, H), lambda i: (0, 0)),
                    pl.BlockSpec((1, H), lambda i: (0, 0))],
        out_specs=pl.BlockSpec((BM, H), lambda i: (i, 0)),
        out_shape=jax.ShapeDtypeStruct((N, H), jnp.float32),
    )(x, mx, sm, *wp)


def _attn_body(h0_ref, h1_ref, h2_ref, p1_ref, b1_ref, o_ref):
    @pl.when(pl.program_id(0) == 0)
    def _():
        o_ref[...] = jnp.zeros_like(o_ref)
    for r, href in enumerate((h0_ref, h1_ref, h2_ref)):
        t = jnp.tanh(jnp.dot(href[...].astype(jnp.bfloat16), p1_ref[...],
                             preferred_element_type=jnp.float32) + b1_ref[...])
        o_ref[pl.ds(r, 1), :] = o_ref[pl.ds(r, 1), :] + jnp.sum(t, axis=0, keepdims=True)


def _tc_attn_colsums(h0, h1, h2, p1_bf16, b1):
    return pl.pallas_call(
        _attn_body,
        grid=(GRID,),
        in_specs=[pl.BlockSpec((BM, H), lambda i: (i, 0)),
                  pl.BlockSpec((BM, H), lambda i: (i, 0)),
                  pl.BlockSpec((BM, H), lambda i: (i, 0)),
                  pl.BlockSpec((H, H), lambda i: (0, 0)),
                  pl.BlockSpec((1, H), lambda i: (0, 0))],
        out_specs=pl.BlockSpec((R, H), lambda i: (0, 0)),
        out_shape=jax.ShapeDtypeStruct((R, H), jnp.float32),
    )(h0, h1, h2, p1_bf16, b1)


def _combine_body(h0_ref, h1_ref, h2_ref, beta_ref, o_ref):
    b = beta_ref[...]
    o_ref[...] = jnp.maximum(
        b[0, 0] * h0_ref[...] + b[0, 1] * h1_ref[...] + b[0, 2] * h2_ref[...],
        0.0)


def _tc_combine_relu(h0, h1, h2, beta):
    return pl.pallas_call(
        _combine_body,
        grid=(GRID,),
        in_specs=[pl.BlockSpec((BM, H), lambda i: (i, 0)),
                  pl.BlockSpec((BM, H), lambda i: (i, 0)),
                  pl.BlockSpec((BM, H), lambda i: (i, 0)),
                  pl.BlockSpec((1, R), lambda i: (0, 0))],
        out_specs=pl.BlockSpec((BM, H), lambda i: (i, 0)),
        out_shape=jax.ShapeDtypeStruct((N, H), jnp.float32),
    )(h0, h1, h2, beta)


def _pred_body(h0_ref, h1_ref, h2_ref, beta_ref, pw_ref, pb_ref, o_ref):
    b = beta_ref[...]
    h = b[0, 0] * h0_ref[...] + b[0, 1] * h1_ref[...] + b[0, 2] * h2_ref[...]
    logit = jnp.sum(h * pw_ref[...], axis=1, keepdims=True) + pb_ref[...]
    o_ref[...] = jax.nn.sigmoid(logit)


def _tc_combine_pred(h0, h1, h2, beta, pw_row, pb):
    return pl.pallas_call(
        _pred_body,
        grid=(GRID,),
        in_specs=[pl.BlockSpec((BM, H), lambda i: (i, 0)),
                  pl.BlockSpec((BM, H), lambda i: (i, 0)),
                  pl.BlockSpec((BM, H), lambda i: (i, 0)),
                  pl.BlockSpec((1, R), lambda i: (0, 0)),
                  pl.BlockSpec((1, H), lambda i: (0, 0)),
                  pl.BlockSpec((1, 1), lambda i: (0, 0))],
        out_specs=pl.BlockSpec((BM, 1), lambda i: (i, 0)),
        out_shape=jax.ShapeDtypeStruct((N, 1), jnp.float32),
    )(h0, h1, h2, beta, pw_row, pb)


def _conv_weights(p):
    w2 = p["fc2_W"]
    return (
        (w2[:H] + w2[H:2 * H]).astype(jnp.bfloat16),
        w2[2 * H:].astype(jnp.bfloat16),
        p["fc2_b"].reshape(1, H),
        p["fc1_W"].astype(jnp.bfloat16),
        p["fc1_b"].reshape(1, H),
        p["fc3_W"][:H].astype(jnp.bfloat16),
        p["fc3_W"][H:].astype(jnp.bfloat16),
        p["fc3_b"].reshape(1, H),
    )


def _layer(x2d, edges, convs, attn_p1, attn_b1, attn_p2):
    hs = []
    for r in range(R):
        mx, sm = _sc_segment_maxsum(x2d, edges[r][0], edges[r][1])
        mx = mx.reshape(NPAD, H)
        sm = sm.reshape(NPAD, H)
        hs.append(_tc_conv(x2d, mx, sm, convs[r]))
    colsums = _tc_attn_colsums(hs[0], hs[1], hs[2], attn_p1, attn_b1)
    w = (colsums @ attn_p2) / N
    beta = jax.nn.softmax(w, axis=0).reshape(1, R)
    return hs, beta


def kernel(x_user, params, edge_index_b0_r0, edge_index_b0_r1, edge_index_b0_r2,
           edge_index_b1_r0, edge_index_b1_r1, edge_index_b1_r2):
    edges0 = [edge_index_b0_r0, edge_index_b0_r1, edge_index_b0_r2]
    edges1 = [edge_index_b1_r0, edge_index_b1_r1, edge_index_b1_r2]

    x = _tc_embed(x_user, params["embed_W"].astype(jnp.bfloat16),
                  params["embed_b"].reshape(1, H))

    convs1 = [_conv_weights(params["conv1_r%d" % r]) for r in range(R)]
    convs2 = [_conv_weights(params["conv2_r%d" % r]) for r in range(R)]
    attn_p1 = params["attn_p1_W"].astype(jnp.bfloat16)
    attn_b1 = params["attn_p1_b"].reshape(1, H)
    attn_p2 = params["attn_p2_W"]

    hs1, beta1 = _layer(x, edges0, convs1, attn_p1, attn_b1, attn_p2)
    h = _tc_combine_relu(hs1[0], hs1[1], hs1[2], beta1)
    hs2, beta2 = _layer(h, edges1, convs2, attn_p1, attn_b1, attn_p2)
    return _tc_combine_pred(hs2[0], hs2[1], hs2[2], beta2,
                            params["pred_W"].reshape(1, H),
                            params["pred_b"].reshape(1, 1))


# speedup vs baseline: 1.1764x; 1.0918x over previous
import functools

import jax
import jax.numpy as jnp
from jax import lax
from jax.experimental import pallas as pl
from jax.experimental.pallas import tpu as pltpu
from jax.experimental.pallas import tpu_sc as plsc

N = 10000
E = 160000
H = 512
R = 3

RN = 105
NRANGES = 96
RPT = 3
NPAD = NRANGES * RN
KB = 32
CHUNK = 2000
NCHUNK = E // CHUNK
CVREGS = CHUNK // 16

_sc_mesh = plsc.VectorSubcoreMesh(core_axis_name="c", subcore_axis_name="s")


@functools.partial(
    pl.kernel,
    mesh=_sc_mesh,
    out_type=[
        jax.ShapeDtypeStruct((NPAD * H,), jnp.float32),
        jax.ShapeDtypeStruct((NPAD * H,), jnp.float32),
    ],
    scratch_types=[
        pltpu.VMEM((RN * H,), jnp.float32),
        pltpu.VMEM((RN * H,), jnp.float32),
        pltpu.VMEM((KB, H), jnp.float32),
        pltpu.VMEM((KB + 16,), jnp.int32),
        pltpu.VMEM((KB + 16,), jnp.int32),
        pltpu.VMEM((KB,), jnp.int32),
        pltpu.VMEM((KB + 16,), jnp.int32),
        pltpu.VMEM((CHUNK,), jnp.int32),
        pltpu.VMEM((CHUNK,), jnp.int32),
        pltpu.SemaphoreType.DMA,
    ],
    compiler_params=pltpu.CompilerParams(needs_layout_passes=False),
)
def _sc_segment_maxsum(x_hbm, src_hbm, dst_hbm, mx_hbm, sm_hbm,
                       accmx, accsm, rows, psrc, pldst, gidx, lsnap,
                       csrc, cdst, gsem):
    wid = lax.axis_index("s") * 2 + lax.axis_index("c")
    lanes = lax.iota(jnp.int32, 16)

    for j in range((KB + 16) // 16):
        psrc[pl.ds(j * 16, 16)] = lax.shift_left(lanes + j * 16, 7)
    for j in range(KB // 16):
        gidx[pl.ds(j * 16, 16)] = lanes + j * 16
        lsnap[pl.ds(j * 16, 16)] = lanes

    def maxsum_edges(nedges, dyn):
        def edge_body(e, carry):
            off = lsnap[pl.ds(e, 16)][0] * H
            for c in range(H // 16):
                rv = rows[e, pl.ds(c * 16, 16)]
                f = c * 16
                mv = accmx[pl.ds(off + f, 16)]
                accmx[pl.ds(off + f, 16)] = jnp.maximum(mv, rv)
                plsc.addupdate_scatter(accsm, [lanes + (off + f)], rv)
            return carry
        if dyn:
            lax.fori_loop(0, nedges, edge_body, 0)
        else:
            lax.fori_loop(0, KB, edge_body, 0, unroll=2)

    def drain_pending():
        pltpu.make_async_copy(x_hbm.at[gidx], rows, gsem).wait()
        maxsum_edges(KB, dyn=False)

    def snapshot(self_=None):
        for j in range(KB // 16):
            v = psrc[pl.ds(j * 16, 16)]
            gidx[pl.ds(j * 16, 16)] = lax.shift_right_logical(v, 7)
            lsnap[pl.ds(j * 16, 16)] = v & 127

    def flush(carry):
        n, fc = carry
        lax.cond(fc > 0, drain_pending, lambda: None)
        snapshot()
        pltpu.async_copy(x_hbm.at[gidx], rows, gsem)
        psrc[pl.ds(0, 16)] = psrc[pl.ds(KB, 16)]
        return (n - KB, fc + 1)

    for k in range(RPT):
        lo = (wid * RPT + k) * RN

        def init_body(i, carry):
            accmx[pl.ds(i * 16, 16)] = jnp.full((16,), -jnp.inf, jnp.float32)
            accsm[pl.ds(i * 16, 16)] = jnp.zeros((16,), jnp.float32)
            return carry
        lax.fori_loop(0, RN * H // 16, init_body, 0)

        def superchunk(s, carry):
            pltpu.sync_copy(src_hbm.at[pl.ds(s * CHUNK, CHUNK)], csrc)
            pltpu.sync_copy(dst_hbm.at[pl.ds(s * CHUNK, CHUNK)], cdst)

            def chunk(j, carry):
                n, fc = carry
                s16 = csrc[pl.ds(j * 16, 16)]
                ld = cdst[pl.ds(j * 16, 16)] - lo
                m = (ld >= 0) & (ld < RN)
                plsc.store_compressed(
                    psrc.at[pl.ds(n, 16)],
                    lax.shift_left(s16, 7) + ld, mask=m)
                n = n + plsc.all_reduce_population_count(m)[0]
                return lax.cond(n >= KB, flush, lambda c: c, (n, fc))

            return lax.fori_loop(0, CVREGS, chunk, carry, unroll=2)

        n, fc = lax.fori_loop(0, NCHUNK, superchunk, (0, 0))
        lax.cond(fc > 0, drain_pending, lambda: None)

        snapshot()
        pltpu.async_copy(x_hbm.at[gidx], rows, gsem).wait()
        maxsum_edges(n, dyn=True)

        def fin_body(i, carry):
            v = accmx[pl.ds(i * 16, 16)]
            accmx[pl.ds(i * 16, 16)] = jnp.where(
                v == -jnp.inf, jnp.zeros((16,), jnp.float32), v)
            return carry
        lax.fori_loop(0, RN * H // 16, fin_body, 0)

        pltpu.sync_copy(accmx, mx_hbm.at[pl.ds(lo * H, RN * H)])
        pltpu.sync_copy(accsm, sm_hbm.at[pl.ds(lo * H, RN * H)])


BM = 400
GRID = N // BM


def _embed_body(xu_ref, w_ref, b_ref, o_ref):
    o_ref[...] = jnp.dot(xu_ref[...].astype(jnp.bfloat16), w_ref[...],
                         preferred_element_type=jnp.float32) + b_ref[...]


def _tc_embed(xu, w_bf16, b):
    return pl.pallas_call(
        _embed_body,
        grid=(GRID,),
        in_specs=[pl.BlockSpec((BM, H), lambda i: (i, 0)),
                  pl.BlockSpec((H, H), lambda i: (0, 0)),
                  pl.BlockSpec((1, H), lambda i: (0, 0))],
        out_specs=pl.BlockSpec((BM, H), lambda i: (i, 0)),
        out_shape=jax.ShapeDtypeStruct((N, H), jnp.float32),
    )(xu, w_bf16, b)


def _conv_body(x_ref, mx_ref, sm_ref, w2ab_ref, w2c_ref, b2_ref,
               w1_ref, b1_ref, w3a_ref, w3b_ref, b3_ref, o_ref):
    a = (jnp.dot(mx_ref[...].astype(jnp.bfloat16), w2ab_ref[...],
                 preferred_element_type=jnp.float32)
         + jnp.dot(sm_ref[...].astype(jnp.bfloat16), w2c_ref[...],
                   preferred_element_type=jnp.float32)
         + b2_ref[...])
    bb = jnp.dot(x_ref[...].astype(jnp.bfloat16), w1_ref[...],
                 preferred_element_type=jnp.float32) + b1_ref[...]
    a = jnp.maximum(a, 0.0).astype(jnp.bfloat16)
    bb = jnp.maximum(bb, 0.0).astype(jnp.bfloat16)
    h = (jnp.dot(a, w3a_ref[...], preferred_element_type=jnp.float32)
         + jnp.dot(bb, w3b_ref[...], preferred_element_type=jnp.float32)
         + b3_ref[...])
    o_ref[...] = jnp.maximum(h, 0.0)


def _tc_conv(x, mx, sm, wp):
    return pl.pallas_call(
        _conv_body,
        grid=(GRID,),
        in_specs=[pl.BlockSpec((BM, H), lambda i: (i, 0)),
                  pl.BlockSpec((BM, H), lambda i: (i, 0)),
                  pl.BlockSpec((BM, H), lambda i: (i, 0))]
                 + [pl.BlockSpec((H, H), lambda i: (0, 0)),
                    pl.BlockSpec((H, H), lambda i: (0, 0)),
                    pl.BlockSpec((1, H), lambda i: (0, 0)),
                    pl.BlockSpec((H, H), lambda i: (0, 0)),
                    pl.BlockSpec((1, H), lambda i: (0, 0)),
                    pl.BlockSpec((H, H), lambda i: (0, 0)),
                    pl.BlockSpec((H, H), lambda i: (0, 0)),
                    pl.BlockSpec((1, H), lambda i: (0, 0))],
        out_specs=pl.BlockSpec((BM, H), lambda i: (i, 0)),
        out_shape=jax.ShapeDtypeStruct((N, H), jnp.float32),
    )(x, mx, sm, *wp)


def _attn_body(h0_ref, h1_ref, h2_ref, p1_ref, b1_ref, o_ref):
    @pl.when(pl.program_id(0) == 0)
    def _():
        o_ref[...] = jnp.zeros_like(o_ref)
    for r, href in enumerate((h0_ref, h1_ref, h2_ref)):
        t = jnp.tanh(jnp.dot(href[...].astype(jnp.bfloat16), p1_ref[...],
                             preferred_element_type=jnp.float32) + b1_ref[...])
        o_ref[pl.ds(r, 1), :] = o_ref[pl.ds(r, 1), :] + jnp.sum(t, axis=0, keepdims=True)


def _tc_attn_colsums(h0, h1, h2, p1_bf16, b1):
    return pl.pallas_call(
        _attn_body,
        grid=(GRID,),
        in_specs=[pl.BlockSpec((BM, H), lambda i: (i, 0)),
                  pl.BlockSpec((BM, H), lambda i: (i, 0)),
                  pl.BlockSpec((BM, H), lambda i: (i, 0)),
                  pl.BlockSpec((H, H), lambda i: (0, 0)),
                  pl.BlockSpec((1, H), lambda i: (0, 0))],
        out_specs=pl.BlockSpec((R, H), lambda i: (0, 0)),
        out_shape=jax.ShapeDtypeStruct((R, H), jnp.float32),
    )(h0, h1, h2, p1_bf16, b1)


def _combine_body(h0_ref, h1_ref, h2_ref, beta_ref, o_ref):
    b = beta_ref[...]
    o_ref[...] = jnp.maximum(
        b[0, 0] * h0_ref[...] + b[0, 1] * h1_ref[...] + b[0, 2] * h2_ref[...],
        0.0)


def _tc_combine_relu(h0, h1, h2, beta):
    return pl.pallas_call(
        _combine_body,
        grid=(GRID,),
        in_specs=[pl.BlockSpec((BM, H), lambda i: (i, 0)),
                  pl.BlockSpec((BM, H), lambda i: (i, 0)),
                  pl.BlockSpec((BM, H), lambda i: (i, 0)),
                  pl.BlockSpec((1, R), lambda i: (0, 0))],
        out_specs=pl.BlockSpec((BM, H), lambda i: (i, 0)),
        out_shape=jax.ShapeDtypeStruct((N, H), jnp.float32),
    )(h0, h1, h2, beta)


def _pred_body(h0_ref, h1_ref, h2_ref, beta_ref, pw_ref, pb_ref, o_ref):
    b = beta_ref[...]
    h = b[0, 0] * h0_ref[...] + b[0, 1] * h1_ref[...] + b[0, 2] * h2_ref[...]
    logit = jnp.sum(h * pw_ref[...], axis=1, keepdims=True) + pb_ref[...]
    o_ref[...] = jax.nn.sigmoid(logit)


def _tc_combine_pred(h0, h1, h2, beta, pw_row, pb):
    return pl.pallas_call(
        _pred_body,
        grid=(GRID,),
        in_specs=[pl.BlockSpec((BM, H), lambda i: (i, 0)),
                  pl.BlockSpec((BM, H), lambda i: (i, 0)),
                  pl.BlockSpec((BM, H), lambda i: (i, 0)),
                  pl.BlockSpec((1, R), lambda i: (0, 0)),
                  pl.BlockSpec((1, H), lambda i: (0, 0)),
                  pl.BlockSpec((1, 1), lambda i: (0, 0))],
        out_specs=pl.BlockSpec((BM, 1), lambda i: (i, 0)),
        out_shape=jax.ShapeDtypeStruct((N, 1), jnp.float32),
    )(h0, h1, h2, beta, pw_row, pb)


def _conv_weights(p):
    w2 = p["fc2_W"]
    return (
        (w2[:H] + w2[H:2 * H]).astype(jnp.bfloat16),
        w2[2 * H:].astype(jnp.bfloat16),
        p["fc2_b"].reshape(1, H),
        p["fc1_W"].astype(jnp.bfloat16),
        p["fc1_b"].reshape(1, H),
        p["fc3_W"][:H].astype(jnp.bfloat16),
        p["fc3_W"][H:].astype(jnp.bfloat16),
        p["fc3_b"].reshape(1, H),
    )


def _layer(x2d, edges, convs, attn_p1, attn_b1, attn_p2):
    hs = []
    for r in range(R):
        mx, sm = _sc_segment_maxsum(x2d, edges[r][0], edges[r][1])
        mx = mx.reshape(NPAD, H)
        sm = sm.reshape(NPAD, H)
        hs.append(_tc_conv(x2d, mx, sm, convs[r]))
    colsums = _tc_attn_colsums(hs[0], hs[1], hs[2], attn_p1, attn_b1)
    w = (colsums @ attn_p2) / N
    beta = jax.nn.softmax(w, axis=0).reshape(1, R)
    return hs, beta


def kernel(x_user, params, edge_index_b0_r0, edge_index_b0_r1, edge_index_b0_r2,
           edge_index_b1_r0, edge_index_b1_r1, edge_index_b1_r2):
    edges0 = [edge_index_b0_r0, edge_index_b0_r1, edge_index_b0_r2]
    edges1 = [edge_index_b1_r0, edge_index_b1_r1, edge_index_b1_r2]

    x = _tc_embed(x_user, params["embed_W"].astype(jnp.bfloat16),
                  params["embed_b"].reshape(1, H))

    convs1 = [_conv_weights(params["conv1_r%d" % r]) for r in range(R)]
    convs2 = [_conv_weights(params["conv2_r%d" % r]) for r in range(R)]
    attn_p1 = params["attn_p1_W"].astype(jnp.bfloat16)
    attn_b1 = params["attn_p1_b"].reshape(1, H)
    attn_p2 = params["attn_p2_W"]

    hs1, beta1 = _layer(x, edges0, convs1, attn_p1, attn_b1, attn_p2)
    h = _tc_combine_relu(hs1[0], hs1[1], hs1[2], beta1)
    hs2, beta2 = _layer(h, edges1, convs2, attn_p1, attn_b1, attn_p2)
    return _tc_combine_pred(hs2[0], hs2[1], hs2[2], beta2,
                            params["pred_W"].reshape(1, H),
                            params["pred_b"].reshape(1, 1))
